# initial kernel scaffold (unmeasured)
import jax
import jax.numpy as jnp
from jax import lax
from jax.experimental import pallas as pl
from jax.experimental.pallas import tpu as pltpu

N_DEV = 8


def kernel(x, w_mat):
    m, k_shard = x.shape
    k, n = w_mat.shape
    m_blk = m // N_DEV
    k_blk = k // N_DEV

    def body(x_ref, w_ref, out_ref, recv_buf, amax_send, amax_recv,
             send_sems, recv_sems, a_send_sems, a_recv_sems):
        me = lax.axis_index("i")

        barrier = pltpu.get_barrier_semaphore()
        for off in range(1, N_DEV):
            dst = lax.rem(me + off, N_DEV)
            pl.semaphore_signal(barrier, inc=1, device_id=(dst,),
                                device_id_type=pl.DeviceIdType.MESH)
        pl.semaphore_wait(barrier, N_DEV - 1)

        sends = []
        for off in range(1, N_DEV):
            dst = lax.rem(me + off, N_DEV)
            rdma = pltpu.make_async_remote_copy(
                src_ref=x_ref.at[pl.ds(dst * m_blk, m_blk), :],
                dst_ref=recv_buf.at[off - 1],
                send_sem=send_sems.at[off - 1],
                recv_sem=recv_sems.at[off - 1],
                device_id=(dst,),
                device_id_type=pl.DeviceIdType.MESH,
            )
            rdma.start()
            sends.append(rdma)

        acc = jnp.dot(
            x_ref[pl.ds(me * m_blk, m_blk), :],
            w_ref[pl.ds(me * k_blk, k_blk), :],
            preferred_element_type=jnp.float32,
        )

        for off in range(1, N_DEV):
            src = lax.rem(me - off + N_DEV, N_DEV)
            sends[off - 1].wait_recv()
            acc = acc + jnp.dot(
                recv_buf[off - 1],
                w_ref[pl.ds(src * k_blk, k_blk), :],
                preferred_element_type=jnp.float32,
            )

        y = jnp.maximum(acc, 0.0)

        local_max = jnp.max(y)
        amax_send[0, :] = jnp.full((128,), local_max, jnp.float32)
        for off in range(1, N_DEV):
            dst = lax.rem(me + off, N_DEV)
            a_rdma = pltpu.make_async_remote_copy(
                src_ref=amax_send,
                dst_ref=amax_recv.at[off - 1],
                send_sem=a_send_sems.at[off - 1],
                recv_sem=a_recv_sems.at[off - 1],
                device_id=(dst,),
                device_id_type=pl.DeviceIdType.MESH,
            )
            a_rdma.start()
            sends.append(a_rdma)

        gmax = local_max
        for off in range(1, N_DEV):
            sends[N_DEV - 1 + off - 1].wait_recv()
            gmax = jnp.maximum(gmax, amax_recv[off - 1, 0, 0])

        for r in sends:
            r.wait_send()

        scale = gmax / 127.0
        q = jnp.clip(jnp.round(y / scale), -127.0, 127.0)
        out_ref[...] = q * scale

    return pl.pallas_call(
        body,
        out_shape=jax.ShapeDtypeStruct((m_blk, n), jnp.float32),
        in_specs=[
            pl.BlockSpec(memory_space=pltpu.VMEM),
            pl.BlockSpec(memory_space=pltpu.VMEM),
        ],
        out_specs=pl.BlockSpec(memory_space=pltpu.VMEM),
        scratch_shapes=[
            pltpu.VMEM((N_DEV - 1, m_blk, k_shard), jnp.float32),
            pltpu.VMEM((1, 128), jnp.float32),
            pltpu.VMEM((N_DEV - 1, 1, 128), jnp.float32),
            pltpu.SemaphoreType.DMA((N_DEV - 1,)),
            pltpu.SemaphoreType.DMA((N_DEV - 1,)),
            pltpu.SemaphoreType.DMA((N_DEV - 1,)),
            pltpu.SemaphoreType.DMA((N_DEV - 1,)),
        ],
        compiler_params=pltpu.CompilerParams(collective_id=0),
    )(x, w_mat)


# baseline (device time: 85840 ns/iter reference)
import jax
import jax.numpy as jnp
from jax import lax
from jax.experimental import pallas as pl
from jax.experimental.pallas import tpu as pltpu

N_DEV = 8


def kernel(x, w_mat):
    m, k_shard = x.shape
    k, n = w_mat.shape
    m_blk = m // N_DEV
    k_blk = k // N_DEV

    def body(x_ref, w_hbm, out_ref, w_buf, recv_buf, amax_send, amax_recv,
             w_sems, send_sems, recv_sems, a_send_sems, a_recv_sems):
        me = lax.axis_index("i")

        barrier = pltpu.get_barrier_semaphore()
        for off in range(1, N_DEV):
            dst = lax.rem(me + off, N_DEV)
            pl.semaphore_signal(barrier, inc=1, device_id=(dst,),
                                device_id_type=pl.DeviceIdType.MESH)
        pl.semaphore_wait(barrier, N_DEV - 1)

        sends = []
        for off in range(1, N_DEV):
            dst = lax.rem(me + off, N_DEV)
            rdma = pltpu.make_async_remote_copy(
                src_ref=x_ref.at[pl.ds(dst * m_blk, m_blk), :],
                dst_ref=recv_buf.at[off - 1],
                send_sem=send_sems.at[off - 1],
                recv_sem=recv_sems.at[off - 1],
                device_id=(dst,),
                device_id_type=pl.DeviceIdType.MESH,
            )
            rdma.start()
            sends.append(rdma)

        def w_copy(step, slot):
            src = lax.rem(me - step + N_DEV, N_DEV)
            return pltpu.make_async_copy(
                w_hbm.at[pl.ds(src * k_blk, k_blk), :],
                w_buf.at[slot],
                w_sems.at[slot],
            )

        w_copy(0, 0).start()
        for s in range(N_DEV):
            slot = s % 2
            w_copy(s, slot).wait()
            if s + 1 < N_DEV:
                w_copy(s + 1, (s + 1) % 2).start()
            if s == 0:
                a_mat = x_ref[pl.ds(me * m_blk, m_blk), :]
            else:
                sends[s - 1].wait_recv()
                a_mat = recv_buf[s - 1]
            partial = jnp.dot(a_mat, w_buf[slot],
                              preferred_element_type=jnp.float32)
            if s == 0:
                out_ref[...] = partial
            else:
                out_ref[...] = out_ref[...] + partial

        out_ref[...] = jnp.maximum(out_ref[...], 0.0)
        local_max = jnp.max(out_ref[...])

        amax_send[0, :] = jnp.full((128,), local_max, jnp.float32)
        for off in range(1, N_DEV):
            dst = lax.rem(me + off, N_DEV)
            a_rdma = pltpu.make_async_remote_copy(
                src_ref=amax_send,
                dst_ref=amax_recv.at[off - 1],
                send_sem=a_send_sems.at[off - 1],
                recv_sem=a_recv_sems.at[off - 1],
                device_id=(dst,),
                device_id_type=pl.DeviceIdType.MESH,
            )
            a_rdma.start()
            sends.append(a_rdma)

        gmax = local_max
        for off in range(1, N_DEV):
            sends[N_DEV - 1 + off - 1].wait_recv()
            gmax = jnp.maximum(gmax, amax_recv[off - 1, 0, 0])

        for r in sends:
            r.wait_send()

        scale = gmax / 127.0
        q = jnp.clip(jnp.round(out_ref[...] / scale), -127.0, 127.0)
        out_ref[...] = q * scale

    return pl.pallas_call(
        body,
        out_shape=jax.ShapeDtypeStruct((m_blk, n), jnp.float32),
        in_specs=[
            pl.BlockSpec(memory_space=pltpu.VMEM),
            pl.BlockSpec(memory_space=pltpu.MemorySpace.HBM),
        ],
        out_specs=pl.BlockSpec(memory_space=pltpu.VMEM),
        scratch_shapes=[
            pltpu.VMEM((2, k_blk, n), jnp.float32),
            pltpu.VMEM((N_DEV - 1, m_blk, k_shard), jnp.float32),
            pltpu.VMEM((1, 128), jnp.float32),
            pltpu.VMEM((N_DEV - 1, 1, 128), jnp.float32),
            pltpu.SemaphoreType.DMA((2,)),
            pltpu.SemaphoreType.DMA((N_DEV - 1,)),
            pltpu.SemaphoreType.DMA((N_DEV - 1,)),
            pltpu.SemaphoreType.DMA((N_DEV - 1,)),
            pltpu.SemaphoreType.DMA((N_DEV - 1,)),
        ],
        compiler_params=pltpu.CompilerParams(
            collective_id=0,
            vmem_limit_bytes=60 * 1024 * 1024,
        ),
    )(x, w_mat)
